# baseline (speedup 1.0000x reference)
import functools

import jax
import jax.numpy as jnp
from jax import lax
from jax.experimental import pallas as pl
from jax.experimental.pallas import tpu as pltpu
from jax.experimental.pallas import tpu_sc as plsc

MODEL_DIM = 4096
NUM_EXPERTS = 64
K = 2
NUM_TOKENS = 8192
BLK = 512
NBLK = NUM_TOKENS // BLK


def _dot(a, b):
    return jax.lax.dot_general(
        a, b, (((1,), (0,)), ((), ())),
        precision=jax.lax.Precision.DEFAULT,
        preferred_element_type=jnp.float32)


def _gate_tc_body(x_ref, wt_ref, ltri_ref, probs_ref,
                  e1_ref, e2_ref, rw1_ref, rw2_ref, cw1_ref, cw2_ref,
                  counts_ref, offsets_ref, carry):
    i = pl.program_id(0)

    @pl.when(i == 0)
    def _init():
        carry[...] = jnp.zeros_like(carry)

    logits = _dot(x_ref[...], wt_ref[...])

    m = jnp.max(logits, axis=-1, keepdims=True)
    el = jnp.exp(logits - m)
    z = jnp.sum(el, axis=-1, keepdims=True)
    probs = el / z
    probs_ref[...] = probs

    iota = lax.broadcasted_iota(jnp.int32, (BLK, NUM_EXPERTS), 1)
    is1 = logits == m
    i1 = jnp.min(jnp.where(is1, iota, NUM_EXPERTS), axis=-1, keepdims=True)
    sel1 = iota == i1
    masked = jnp.where(sel1, -jnp.inf, logits)
    m2 = jnp.max(masked, axis=-1, keepdims=True)
    is2 = masked == m2
    i2 = jnp.min(jnp.where(is2, iota, NUM_EXPERTS), axis=-1, keepdims=True)
    sel2 = iota == i2

    p1 = jnp.sum(jnp.where(sel1, probs, 0.0), axis=-1, keepdims=True)
    p2 = jnp.sum(jnp.where(sel2, probs, 0.0), axis=-1, keepdims=True)
    q = jnp.exp(p2 - p1)
    zz = 1.0 + q
    cw1_ref[...] = (1.0 / zz).reshape(BLK)
    cw2_ref[...] = (q / zz).reshape(BLK)
    e1_ref[...] = i1.reshape(BLK)
    e2_ref[...] = i2.reshape(BLK)

    occ = (sel1 | sel2).astype(jnp.bfloat16)
    c_local = jnp.dot(ltri_ref[...], occ, preferred_element_type=jnp.float32)
    c_tot = carry[...] + c_local

    r1 = jnp.sum(jnp.where(sel1, c_tot, 0.0), axis=-1, keepdims=True)
    r2 = jnp.sum(jnp.where(sel2, c_tot, 0.0), axis=-1, keepdims=True)
    rw1_ref[...] = r1.astype(jnp.int32).reshape(BLK)
    rw2_ref[...] = r2.astype(jnp.int32).reshape(BLK)

    new_carry = carry[...] + jnp.sum(occ.astype(jnp.float32), axis=0,
                                     keepdims=True)
    carry[...] = new_carry

    @pl.when(i == NBLK - 1)
    def _fin():
        counts = new_carry.astype(jnp.int32)
        counts_ref[...] = counts[0]
        inc = counts
        for sh in (1, 2, 4, 8, 16, 32):
            shifted = jnp.concatenate(
                [jnp.zeros((1, sh), jnp.int32), inc[:, :-sh]], axis=1)
            inc = inc + shifted
        offsets_ref[...] = (inc - counts)[0]


def _tok_spec():
    return pl.BlockSpec((BLK,), lambda i: (i,))


def _gate_tc(x, wt, ltri):
    i32 = jnp.int32
    f32 = jnp.float32
    return pl.pallas_call(
        _gate_tc_body,
        grid=(NBLK,),
        in_specs=[
            pl.BlockSpec((BLK, MODEL_DIM), lambda i: (i, 0)),
            pl.BlockSpec((MODEL_DIM, NUM_EXPERTS), lambda i: (0, 0)),
            pl.BlockSpec((BLK, BLK), lambda i: (0, 0)),
        ],
        out_specs=[
            pl.BlockSpec((BLK, NUM_EXPERTS), lambda i: (i, 0)),
            _tok_spec(), _tok_spec(), _tok_spec(), _tok_spec(),
            _tok_spec(), _tok_spec(),
            pl.BlockSpec((NUM_EXPERTS,), lambda i: (0,)),
            pl.BlockSpec((NUM_EXPERTS,), lambda i: (0,)),
        ],
        out_shape=[
            jax.ShapeDtypeStruct((NUM_TOKENS, NUM_EXPERTS), f32),
            jax.ShapeDtypeStruct((NUM_TOKENS,), i32),
            jax.ShapeDtypeStruct((NUM_TOKENS,), i32),
            jax.ShapeDtypeStruct((NUM_TOKENS,), i32),
            jax.ShapeDtypeStruct((NUM_TOKENS,), i32),
            jax.ShapeDtypeStruct((NUM_TOKENS,), f32),
            jax.ShapeDtypeStruct((NUM_TOKENS,), f32),
            jax.ShapeDtypeStruct((NUM_EXPERTS,), i32),
            jax.ShapeDtypeStruct((NUM_EXPERTS,), i32),
        ],
        scratch_shapes=[pltpu.VMEM((1, NUM_EXPERTS), jnp.float32)],
    )(x, wt, ltri)


N_FLAT = NUM_TOKENS * K
_NUM_SC_CORES = 2
_NUM_SUBCORES = 16
_NW = _NUM_SC_CORES * _NUM_SUBCORES
_TCHUNK = NUM_TOKENS // _NW
_TG = _TCHUNK // 16
_FCHUNK = _TCHUNK * K
_HALF = N_FLAT // _NUM_SC_CORES


def _gate_sc_body(e1_hbm, e2_hbm, rw1_hbm, rw2_hbm, cw1_hbm, cw2_hbm,
                  offs_hbm, rev_hbm, div_hbm, cwi_hbm,
                  e1_v, e2_v, rw1_v, rw2_v, cw1_v, cw2_v, offs_v,
                  rev_v, cwi_v, ef1_v, ef2_v, rwf1_v, rwf2_v, div_v,
                  sem, sem_out, sem_scan):
    cid = lax.axis_index("c")
    sid = lax.axis_index("s")
    wid = sid * _NUM_SC_CORES + cid
    t0 = wid * _TCHUNK
    tsl = pl.ds(t0, _TCHUNK)
    in_copies = [
        pltpu.async_copy(e1_hbm.at[tsl], e1_v, sem),
        pltpu.async_copy(e2_hbm.at[tsl], e2_v, sem),
        pltpu.async_copy(rw1_hbm.at[tsl], rw1_v, sem),
        pltpu.async_copy(rw2_hbm.at[tsl], rw2_v, sem),
        pltpu.async_copy(cw1_hbm.at[tsl], cw1_v, sem),
        pltpu.async_copy(cw2_hbm.at[tsl], cw2_v, sem),
        pltpu.async_copy(offs_hbm, offs_v, sem),
    ]
    for cp in in_copies:
        cp.wait()
    for g in range(_TG):
        sl = pl.ds(g * 16, 16)
        iv2 = (g * 16 + lax.iota(jnp.int32, 16)) * 2
        r1 = plsc.load_gather(offs_v, [e1_v[sl]]) + rw1_v[sl]
        r2 = plsc.load_gather(offs_v, [e2_v[sl]]) + rw2_v[sl]
        plsc.store_scatter(rev_v, [iv2], r1)
        plsc.store_scatter(rev_v, [iv2 + 1], r2)
        plsc.store_scatter(cwi_v, [iv2], cw1_v[sl])
        plsc.store_scatter(cwi_v, [iv2 + 1], cw2_v[sl])
    fsl = pl.ds(wid * _FCHUNK, _FCHUNK)
    out_copies = [
        pltpu.async_copy(rev_v, rev_hbm.at[fsl], sem_out),
        pltpu.async_copy(cwi_v, cwi_hbm.at[fsl], sem_out),
    ]

    @pl.when(sid == 0)
    def _scatter():
        cps = [
            pltpu.async_copy(e1_hbm, ef1_v, sem_scan),
            pltpu.async_copy(e2_hbm, ef2_v, sem_scan),
            pltpu.async_copy(rw1_hbm, rwf1_v, sem_scan),
            pltpu.async_copy(rw2_hbm, rwf2_v, sem_scan),
        ]
        for cp in cps:
            cp.wait()
        lo = cid * _HALF

        @pl.loop(0, NUM_TOKENS // 16, unroll=8)
        def _it(g):
            sl = pl.ds(g * 16, 16)
            tok = g * 16 + lax.iota(jnp.int32, 16)
            r1 = plsc.load_gather(offs_v, [ef1_v[sl]]) + rwf1_v[sl]
            r2 = plsc.load_gather(offs_v, [ef2_v[sl]]) + rwf2_v[sl]
            m1 = (r1 >= lo) & (r1 < lo + _HALF)
            m2 = (r2 >= lo) & (r2 < lo + _HALF)
            plsc.store_scatter(div_v, [r1 - lo], tok, mask=m1)
            plsc.store_scatter(div_v, [r2 - lo], tok, mask=m2)

        pltpu.sync_copy(div_v, div_hbm.at[pl.ds(lo, _HALF)])

    for cp in out_copies:
        cp.wait()


@functools.cache
def _build_gate_sc():
    i32 = jnp.int32
    f32 = jnp.float32
    return pl.kernel(
        _gate_sc_body,
        out_type=[
            jax.ShapeDtypeStruct((N_FLAT,), i32),
            jax.ShapeDtypeStruct((N_FLAT,), i32),
            jax.ShapeDtypeStruct((N_FLAT,), f32),
        ],
        mesh=plsc.VectorSubcoreMesh(core_axis_name="c",
                                    subcore_axis_name="s"),
        compiler_params=pltpu.CompilerParams(needs_layout_passes=False),
        scratch_types=[
            pltpu.VMEM((_TCHUNK,), i32),
            pltpu.VMEM((_TCHUNK,), i32),
            pltpu.VMEM((_TCHUNK,), i32),
            pltpu.VMEM((_TCHUNK,), i32),
            pltpu.VMEM((_TCHUNK,), f32),
            pltpu.VMEM((_TCHUNK,), f32),
            pltpu.VMEM((NUM_EXPERTS,), i32),
            pltpu.VMEM((_FCHUNK,), i32),
            pltpu.VMEM((_FCHUNK,), f32),
            pltpu.VMEM((NUM_TOKENS,), i32),
            pltpu.VMEM((NUM_TOKENS,), i32),
            pltpu.VMEM((NUM_TOKENS,), i32),
            pltpu.VMEM((NUM_TOKENS,), i32),
            pltpu.VMEM((_HALF,), i32),
            pltpu.SemaphoreType.DMA,
            pltpu.SemaphoreType.DMA,
            pltpu.SemaphoreType.DMA,
        ],
    )


def kernel(inputs, W):
    wt = W.T
    ltri = (jnp.arange(BLK, dtype=jnp.int32)[None, :]
            < jnp.arange(BLK, dtype=jnp.int32)[:, None]).astype(jnp.bfloat16)
    (probs, e1, e2, rw1, rw2, cw1, cw2, counts,
     offsets) = _gate_tc(inputs, wt, ltri)
    rev, sort_div, cwi = _build_gate_sc()(e1, e2, rw1, rw2, cw1, cw2,
                                          offsets)
    input_splits = counts.astype(jnp.int64)
    return (sort_div, rev, cwi, input_splits, probs)


# speedup vs baseline: 1.2106x; 1.2106x over previous
import functools

import jax
import jax.numpy as jnp
from jax import lax
from jax.experimental import pallas as pl
from jax.experimental.pallas import tpu as pltpu
from jax.experimental.pallas import tpu_sc as plsc

MODEL_DIM = 4096
NUM_EXPERTS = 64
K = 2
NUM_TOKENS = 8192
BLK = 512
NBLK = NUM_TOKENS // BLK


def _gate_tc_body(x_ref, wt_ref, ltri_ref, probs_ref, cw_ref, eidx_ref,
                  rwe_ref, counts_ref, offsets_ref, carry):
    i = pl.program_id(0)

    @pl.when(i == 0)
    def _init():
        carry[...] = jnp.zeros_like(carry)

    logits = jax.lax.dot_general(
        x_ref[...], wt_ref[...], (((1,), (0,)), ((), ())),
        precision=jax.lax.Precision.DEFAULT,
        preferred_element_type=jnp.float32)

    m = jnp.max(logits, axis=-1, keepdims=True)
    el = jnp.exp(logits - m)
    z = jnp.sum(el, axis=-1, keepdims=True)
    probs = el / z
    probs_ref[...] = probs

    iota = lax.broadcasted_iota(jnp.int32, (BLK, NUM_EXPERTS), 1)
    is1 = logits == m
    i1 = jnp.min(jnp.where(is1, iota, NUM_EXPERTS), axis=-1, keepdims=True)
    sel1 = iota == i1
    masked = jnp.where(sel1, -jnp.inf, logits)
    m2 = jnp.max(masked, axis=-1, keepdims=True)
    is2 = masked == m2
    i2 = jnp.min(jnp.where(is2, iota, NUM_EXPERTS), axis=-1, keepdims=True)
    sel2 = iota == i2

    p1 = jnp.sum(jnp.where(sel1, probs, 0.0), axis=-1, keepdims=True)
    p2 = jnp.sum(jnp.where(sel2, probs, 0.0), axis=-1, keepdims=True)
    q = jnp.exp(p2 - p1)
    zz = 1.0 + q
    cw_ref[...] = jnp.concatenate([1.0 / zz, q / zz], axis=1)
    eidx_ref[...] = jnp.concatenate([i1, i2], axis=1)

    occ = (sel1 | sel2).astype(jnp.bfloat16)
    c_local = jnp.dot(ltri_ref[...], occ, preferred_element_type=jnp.float32)
    c_tot = carry[...] + c_local

    r1 = jnp.sum(jnp.where(sel1, c_tot, 0.0), axis=-1, keepdims=True)
    r2 = jnp.sum(jnp.where(sel2, c_tot, 0.0), axis=-1, keepdims=True)
    rwe_ref[...] = jnp.concatenate([r1, r2], axis=1).astype(jnp.int32)

    new_carry = carry[...] + jnp.sum(occ.astype(jnp.float32), axis=0,
                                     keepdims=True)
    carry[...] = new_carry

    @pl.when(i == NBLK - 1)
    def _fin():
        counts = new_carry.astype(jnp.int32)
        counts_ref[...] = counts[0]
        inc = counts
        for sh in (1, 2, 4, 8, 16, 32):
            shifted = jnp.concatenate(
                [jnp.zeros((1, sh), jnp.int32), inc[:, :-sh]], axis=1)
            inc = inc + shifted
        offsets_ref[...] = (inc - counts)[0]


def _gate_tc(x, wt, ltri):
    return pl.pallas_call(
        _gate_tc_body,
        grid=(NBLK,),
        in_specs=[
            pl.BlockSpec((BLK, MODEL_DIM), lambda i: (i, 0)),
            pl.BlockSpec((MODEL_DIM, NUM_EXPERTS), lambda i: (0, 0)),
            pl.BlockSpec((BLK, BLK), lambda i: (0, 0)),
        ],
        out_specs=[
            pl.BlockSpec((BLK, NUM_EXPERTS), lambda i: (i, 0)),
            pl.BlockSpec((BLK, K), lambda i: (i, 0)),
            pl.BlockSpec((BLK, K), lambda i: (i, 0)),
            pl.BlockSpec((BLK, K), lambda i: (i, 0)),
            pl.BlockSpec((NUM_EXPERTS,), lambda i: (0,)),
            pl.BlockSpec((NUM_EXPERTS,), lambda i: (0,)),
        ],
        out_shape=[
            jax.ShapeDtypeStruct((NUM_TOKENS, NUM_EXPERTS), jnp.float32),
            jax.ShapeDtypeStruct((NUM_TOKENS, K), jnp.float32),
            jax.ShapeDtypeStruct((NUM_TOKENS, K), jnp.int32),
            jax.ShapeDtypeStruct((NUM_TOKENS, K), jnp.int32),
            jax.ShapeDtypeStruct((NUM_EXPERTS,), jnp.int32),
            jax.ShapeDtypeStruct((NUM_EXPERTS,), jnp.int32),
        ],
        scratch_shapes=[pltpu.VMEM((1, NUM_EXPERTS), jnp.float32)],
    )(x, wt, ltri)


N_FLAT = NUM_TOKENS * K
_NUM_SC_CORES = 2
_NUM_SUBCORES = 16
_NW = _NUM_SC_CORES * _NUM_SUBCORES
_CHUNK = N_FLAT // _NW
_NVEC = _CHUNK // 16
_HALF = N_FLAT // _NUM_SC_CORES


def _gate_sc_body(eidx_hbm, rwe_hbm, offs_hbm, rev_hbm, div_hbm,
                  e_v, rwe_v, offs_v, rev_v, ef_v, rwf_v, div_v, sem):
    cid = lax.axis_index("c")
    sid = lax.axis_index("s")
    wid = sid * _NUM_SC_CORES + cid
    base = wid * _CHUNK
    in_copies = [
        pltpu.async_copy(eidx_hbm.at[pl.ds(base, _CHUNK)], e_v, sem),
        pltpu.async_copy(rwe_hbm.at[pl.ds(base, _CHUNK)], rwe_v, sem),
        pltpu.async_copy(offs_hbm, offs_v, sem),
    ]
    for cp in in_copies:
        cp.wait()
    for j in range(_NVEC):
        sl = pl.ds(j * 16, 16)
        e = e_v[sl]
        rw = rwe_v[sl]
        off = plsc.load_gather(offs_v, [e])
        r = off + rw
        rev_v[sl] = r
    rev_cp = pltpu.async_copy(rev_v, rev_hbm.at[pl.ds(base, _CHUNK)], sem)

    @pl.when(sid == 0)
    def _scatter():
        cps = [
            pltpu.async_copy(eidx_hbm, ef_v, sem),
            pltpu.async_copy(rwe_hbm, rwf_v, sem),
        ]
        for cp in cps:
            cp.wait()
        lo = cid * _HALF

        @plsc.parallel_loop(0, N_FLAT // 16, unroll=8)
        def _it(j):
            sl = pl.ds(j * 16, 16)
            e = ef_v[sl]
            rw = rwf_v[sl]
            r = plsc.load_gather(offs_v, [e]) + rw
            val = (j * 16 + lax.iota(jnp.int32, 16)) >> 1
            m = (r >= lo) & (r < lo + _HALF)
            plsc.store_scatter(div_v, [r - lo], val, mask=m)

        pltpu.sync_copy(div_v, div_hbm.at[pl.ds(lo, _HALF)])

    rev_cp.wait()


@functools.cache
def _build_gate_sc():
    return pl.kernel(
        _gate_sc_body,
        out_type=[
            jax.ShapeDtypeStruct((N_FLAT,), jnp.int32),
            jax.ShapeDtypeStruct((N_FLAT,), jnp.int32),
        ],
        mesh=plsc.VectorSubcoreMesh(core_axis_name="c",
                                    subcore_axis_name="s"),
        compiler_params=pltpu.CompilerParams(needs_layout_passes=False),
        scratch_types=[
            pltpu.VMEM((_CHUNK,), jnp.int32),
            pltpu.VMEM((_CHUNK,), jnp.int32),
            pltpu.VMEM((NUM_EXPERTS,), jnp.int32),
            pltpu.VMEM((_CHUNK,), jnp.int32),
            pltpu.VMEM((N_FLAT,), jnp.int32),
            pltpu.VMEM((N_FLAT,), jnp.int32),
            pltpu.VMEM((_HALF,), jnp.int32),
            pltpu.SemaphoreType.DMA,
        ],
    )


def kernel(inputs, W):
    wt = W.T
    ltri = (jnp.arange(BLK, dtype=jnp.int32)[None, :]
            < jnp.arange(BLK, dtype=jnp.int32)[:, None]).astype(jnp.bfloat16)
    probs, cw, eidx, rwe, counts, offsets = _gate_tc(inputs, wt, ltri)
    rev, sort_div = _build_gate_sc()(eidx.reshape(-1), rwe.reshape(-1),
                                     offsets)
    input_splits = counts.astype(jnp.int64)
    return (sort_div, rev, cw.reshape(-1), input_splits, probs)
